# Initial kernel scaffold; baseline (speedup 1.0000x reference)
import jax
import jax.numpy as jnp
from jax.experimental import pallas as pl


def kernel(x, edge_index, W1, b1, W2, b2):
    raise NotImplementedError("write your pallas kernel here")


# speedup vs baseline: 28.9666x; 28.9666x over previous
import functools

import jax
import jax.numpy as jnp
from jax import lax
from jax.experimental import pallas as pl
from jax.experimental.pallas import tpu as pltpu
from jax.experimental.pallas import tpu_sc as plsc

N_NODES = 10000
E_TOTAL = 320000
D_FEAT = 128
D_HID = 16
N_CLASSES = 10

NC, NS, LANES = 2, 16, 16
NW = NC * NS
EPT = E_TOTAL // NW
CHUNK = 80
NCHUNK = EPT // CHUNK
RZ = 624
N_PAD = 10240
DZ = N_PAD // NS

_mesh = plsc.VectorSubcoreMesh(core_axis_name="c", subcore_axis_name="s")


@functools.partial(
    pl.kernel,
    mesh=_mesh,
    out_type=jax.ShapeDtypeStruct((NC, 1, N_PAD), jnp.float32),
    scratch_types=[
        pltpu.VMEM((NCHUNK, CHUNK), jnp.int32),
        pltpu.VMEM((CHUNK,), jnp.float32),
        pltpu.VMEM((DZ,), jnp.float32),
        pltpu.VMEM_SHARED((N_PAD,), jnp.float32),
    ],
)
def _deg_kernel(dst_hbm, out_hbm, idx_v, ones_v, zb_v, acc_sh):
    c = lax.axis_index("c")
    s = lax.axis_index("s")
    wid = s * NC + c
    pltpu.sync_copy(dst_hbm.at[wid], idx_v)
    one16 = jnp.ones((LANES,), jnp.float32)
    zero16 = jnp.zeros((LANES,), jnp.float32)
    for i in range(CHUNK // LANES):
        ones_v[pl.ds(i * LANES, LANES)] = one16
    for i in range(DZ // LANES):
        zb_v[pl.ds(i * LANES, LANES)] = zero16
    pltpu.sync_copy(zb_v, acc_sh.at[pl.ds(s * DZ, DZ)])
    plsc.subcore_barrier()

    def body(j, carry):
        pltpu.sync_copy(ones_v, acc_sh.at[idx_v.at[j]], add=True)
        return carry

    lax.fori_loop(0, NCHUNK, body, 0)
    plsc.subcore_barrier()
    pltpu.sync_copy(acc_sh.at[pl.ds(s * DZ, DZ)],
                    out_hbm.at[c, 0, pl.ds(s * DZ, DZ)])


@functools.partial(
    pl.kernel,
    mesh=_mesh,
    compiler_params=pltpu.CompilerParams(use_tc_tiling_on_sc=False),
    out_type=jax.ShapeDtypeStruct((NC, N_NODES, D_HID), jnp.float32),
    scratch_types=[
        pltpu.VMEM((NCHUNK, CHUNK), jnp.int32),
        pltpu.VMEM((NCHUNK, CHUNK), jnp.int32),
        pltpu.VMEM((CHUNK, D_HID), jnp.float32),
        pltpu.VMEM((RZ, D_HID), jnp.float32),
        pltpu.VMEM_SHARED((N_NODES, D_HID), jnp.float32),
        pltpu.SemaphoreType.DMA,
    ],
)
def _agg_kernel(src_hbm, dst_hbm, g_hbm, out_hbm,
                sidx, didx, rows, zb, acc_sh, sem):
    c = lax.axis_index("c")
    s = lax.axis_index("s")
    wid = s * NC + c
    pltpu.sync_copy(src_hbm.at[wid], sidx)
    pltpu.sync_copy(dst_hbm.at[wid], didx)
    zero16 = jnp.zeros((LANES,), jnp.float32)

    def zbody(i, carry):
        zb[i, :] = zero16
        return carry

    lax.fori_loop(0, RZ, zbody, 0)
    pltpu.sync_copy(zb, acc_sh.at[pl.ds(s * RZ, RZ)])
    tail = N_NODES - NS * RZ

    @pl.when(s == NS - 1)
    def _():
        pltpu.sync_copy(zb.at[pl.ds(0, tail)],
                        acc_sh.at[pl.ds(NS * RZ, tail)])

    plsc.subcore_barrier()

    def body(j, carry):
        pltpu.async_copy(g_hbm.at[sidx.at[j]], rows, sem).wait()
        pltpu.sync_copy(rows, acc_sh.at[didx.at[j]], add=True)
        return carry

    lax.fori_loop(0, NCHUNK, body, 0)
    plsc.subcore_barrier()
    pltpu.sync_copy(acc_sh.at[pl.ds(s * RZ, RZ)],
                    out_hbm.at[c, pl.ds(s * RZ, RZ)])

    @pl.when(s == NS - 1)
    def _():
        pltpu.sync_copy(acc_sh.at[pl.ds(NS * RZ, tail)],
                        out_hbm.at[c, pl.ds(NS * RZ, tail)])


def _t1_body(x_ref, w1_ref, degp_ref, g1_ref, dinv_ref):
    deg = degp_ref[0] + degp_ref[1] + 1.0
    dinv = lax.rsqrt(deg)
    h = jnp.dot(x_ref[...], w1_ref[...], preferred_element_type=jnp.float32)
    g1_ref[...] = h * dinv
    dinv_ref[...] = dinv


_t1_call = pl.pallas_call(
    _t1_body,
    out_shape=[
        jax.ShapeDtypeStruct((N_NODES, D_HID), jnp.float32),
        jax.ShapeDtypeStruct((N_NODES, 1), jnp.float32),
    ],
)


def _t2_body(accp_ref, g1_ref, dinv_ref, b1_ref, w2_ref, g2_ref):
    acc = accp_ref[0] + accp_ref[1]
    dinv = dinv_ref[...]
    z = jnp.maximum(dinv * (acc + g1_ref[...]) + b1_ref[...], 0.0)
    h2 = jnp.dot(z, w2_ref[...], preferred_element_type=jnp.float32)
    g2_ref[...] = h2 * dinv


_t2_call = pl.pallas_call(
    _t2_body,
    out_shape=jax.ShapeDtypeStruct((N_NODES, D_HID), jnp.float32),
)


def _t3_body(accp_ref, g2_ref, dinv_ref, b2_ref, out_ref):
    acc = accp_ref[0] + accp_ref[1]
    y = dinv_ref[...] * (acc + g2_ref[...]) + b2_ref[...]
    col = lax.broadcasted_iota(jnp.int32, (N_NODES, D_HID), 1)
    mask = col < N_CLASSES
    z = jnp.where(mask, y, -1e30)
    m = jnp.max(z, axis=1, keepdims=True)
    e = jnp.where(mask, jnp.exp(z - m), 0.0)
    ssum = jnp.sum(e, axis=1, keepdims=True)
    out_ref[...] = z - (m + jnp.log(ssum))


_t3_call = pl.pallas_call(
    _t3_body,
    out_shape=jax.ShapeDtypeStruct((N_NODES, D_HID), jnp.float32),
)


def kernel(x, edge_index, W1, b1, W2, b2):
    ei = edge_index.astype(jnp.int32)
    src = ei[0].reshape(NW, NCHUNK, CHUNK)
    dst = ei[1].reshape(NW, NCHUNK, CHUNK)
    degp = _deg_kernel(dst)
    degp = degp.reshape(NC, N_PAD)[:, :N_NODES].reshape(NC, N_NODES, 1)
    g1, dinv = _t1_call(x, W1, degp)
    accp1 = _agg_kernel(src, dst, g1)
    w2p = jnp.pad(W2, ((0, 0), (0, D_HID - N_CLASSES)))
    g2 = _t2_call(accp1, g1, dinv, b1.reshape(1, D_HID), w2p)
    accp2 = _agg_kernel(src, dst, g2)
    b2p = jnp.pad(b2, (0, D_HID - N_CLASSES)).reshape(1, D_HID)
    out16 = _t3_call(accp2, g2, dinv, b2p)
    return out16[:, :N_CLASSES]


# speedup vs baseline: 42.0664x; 1.4522x over previous
import functools

import jax
import jax.numpy as jnp
from jax import lax
from jax.experimental import pallas as pl
from jax.experimental.pallas import tpu as pltpu
from jax.experimental.pallas import tpu_sc as plsc

N_NODES = 10000
E_TOTAL = 320000
D_FEAT = 128
D_HID = 16
N_CLASSES = 10

NC, NS, LANES = 2, 16, 16
NW = NC * NS
EPT = E_TOTAL // NW
CHUNK = 128
NCHUNK = 80
E_TILE = NCHUNK * CHUNK
N_PAD = 10240
DZ = N_PAD // NS
K = 5
NGROUP = NCHUNK // K
NSETS = 3

_mesh = plsc.VectorSubcoreMesh(core_axis_name="c", subcore_axis_name="s")


@functools.partial(
    pl.kernel,
    mesh=_mesh,
    out_type=jax.ShapeDtypeStruct((NC, 1, N_PAD), jnp.float32),
    scratch_types=[
        pltpu.VMEM((NCHUNK, CHUNK), jnp.int32),
        pltpu.VMEM((CHUNK,), jnp.float32),
        pltpu.VMEM((DZ,), jnp.float32),
        pltpu.VMEM_SHARED((N_PAD,), jnp.float32),
    ],
)
def _deg_kernel(dst_hbm, out_hbm, idx_v, ones_v, zb_v, acc_sh):
    c = lax.axis_index("c")
    s = lax.axis_index("s")
    wid = s * NC + c
    pltpu.sync_copy(dst_hbm.at[wid], idx_v)
    one16 = jnp.ones((LANES,), jnp.float32)
    zero16 = jnp.zeros((LANES,), jnp.float32)
    for i in range(CHUNK // LANES):
        ones_v[pl.ds(i * LANES, LANES)] = one16
    for i in range(DZ // LANES):
        zb_v[pl.ds(i * LANES, LANES)] = zero16
    pltpu.sync_copy(zb_v, acc_sh.at[pl.ds(s * DZ, DZ)])
    plsc.subcore_barrier()

    def body(j, carry):
        pltpu.sync_copy(ones_v, acc_sh.at[idx_v.at[j]], add=True)
        return carry

    lax.fori_loop(0, NCHUNK, body, 0)
    plsc.subcore_barrier()
    pltpu.sync_copy(acc_sh.at[pl.ds(s * DZ, DZ)],
                    out_hbm.at[c, 0, pl.ds(s * DZ, DZ)])


@functools.partial(
    pl.kernel,
    mesh=_mesh,
    compiler_params=pltpu.CompilerParams(use_tc_tiling_on_sc=False),
    out_type=jax.ShapeDtypeStruct((NC, N_PAD, D_HID), jnp.float32),
    scratch_types=[
        pltpu.VMEM((NCHUNK, CHUNK), jnp.int32),
        pltpu.VMEM((NCHUNK, CHUNK), jnp.int32),
        pltpu.VMEM((K * CHUNK, D_HID), jnp.float32),
        pltpu.VMEM((K * CHUNK, D_HID), jnp.float32),
        pltpu.VMEM((K * CHUNK, D_HID), jnp.float32),
        pltpu.VMEM((DZ, D_HID), jnp.float32),
        pltpu.VMEM_SHARED((N_PAD, D_HID), jnp.float32),
        pltpu.SemaphoreType.DMA,
        pltpu.SemaphoreType.DMA,
        pltpu.SemaphoreType.DMA,
        pltpu.SemaphoreType.DMA,
        pltpu.SemaphoreType.DMA,
        pltpu.SemaphoreType.DMA,
    ],
)
def _agg_kernel(src_hbm, dst_hbm, g_hbm, out_hbm,
                sidx, didx, buf0, buf1, buf2, zb, acc_sh,
                gsem0, gsem1, gsem2, ssem0, ssem1, ssem2):
    c = lax.axis_index("c")
    s = lax.axis_index("s")
    wid = s * NC + c
    bufs = (buf0, buf1, buf2)
    gsems = (gsem0, gsem1, gsem2)
    ssems = (ssem0, ssem1, ssem2)

    pltpu.sync_copy(src_hbm.at[wid], sidx)
    pltpu.sync_copy(dst_hbm.at[wid], didx)
    zero16 = jnp.zeros((LANES,), jnp.float32)

    def zbody(i, carry):
        zb[i, :] = zero16
        return carry

    lax.fori_loop(0, DZ, zbody, 0)
    pltpu.sync_copy(zb, acc_sh.at[pl.ds(s * DZ, DZ)])
    plsc.subcore_barrier()

    def fire_gathers(g, st):
        for k in range(K):
            pltpu.async_copy(g_hbm.at[sidx.at[g * K + k]],
                             bufs[st].at[pl.ds(k * CHUNK, CHUNK)], gsems[st])

    def wait_gathers(st):
        for _ in range(K):
            pltpu.make_async_copy(g_hbm.at[sidx.at[0]],
                                  bufs[st].at[pl.ds(0, CHUNK)],
                                  gsems[st]).wait()

    def fire_scatters(g, st):
        for k in range(K):
            pltpu.async_copy(bufs[st].at[pl.ds(k * CHUNK, CHUNK)],
                             acc_sh.at[didx.at[g * K + k]], ssems[st],
                             add=True)

    def wait_scatters(st):
        for _ in range(K):
            pltpu.make_async_copy(bufs[st].at[pl.ds(0, CHUNK)],
                                  acc_sh.at[didx.at[0]], ssems[st]).wait()

    def part(g, st, drain_next, fire_next):
        if drain_next:
            wait_scatters((st + 1) % NSETS)
        if fire_next:
            fire_gathers(g + 1, (st + 1) % NSETS)
        wait_gathers(st)
        fire_scatters(g, st)

    fire_gathers(0, 0)
    part(0, 0, drain_next=False, fire_next=True)
    part(1, 1, drain_next=False, fire_next=True)
    part(2, 2, drain_next=True, fire_next=True)

    def gbody(t, carry):
        g = t * NSETS
        part(g + 0, 0, drain_next=True, fire_next=True)
        part(g + 1, 1, drain_next=True, fire_next=True)
        part(g + 2, 2, drain_next=True, fire_next=True)
        return carry

    lax.fori_loop(1, NGROUP // NSETS, gbody, 0)

    part(NGROUP - 1, 0, drain_next=True, fire_next=False)
    wait_scatters(2)
    wait_scatters(0)

    plsc.subcore_barrier()
    pltpu.sync_copy(acc_sh.at[pl.ds(s * DZ, DZ)],
                    out_hbm.at[c, pl.ds(s * DZ, DZ)])


def _t1_body(x_ref, w1_ref, degp_ref, g1_ref, dinv_ref):
    deg = degp_ref[0] + degp_ref[1] + 1.0
    dinv = lax.rsqrt(deg)
    h = jnp.dot(x_ref[...], w1_ref[...], preferred_element_type=jnp.float32)
    g1_ref[...] = h * dinv
    dinv_ref[...] = dinv


_t1_call = pl.pallas_call(
    _t1_body,
    out_shape=[
        jax.ShapeDtypeStruct((N_NODES, D_HID), jnp.float32),
        jax.ShapeDtypeStruct((N_NODES, 1), jnp.float32),
    ],
)


def _t2_body(accp_ref, g1_ref, dinv_ref, b1_ref, w2_ref, g2_ref):
    acc = (accp_ref[0, pl.ds(0, N_NODES), :] +
           accp_ref[1, pl.ds(0, N_NODES), :])
    dinv = dinv_ref[...]
    z = jnp.maximum(dinv * (acc + g1_ref[...]) + b1_ref[...], 0.0)
    h2 = jnp.dot(z, w2_ref[...], preferred_element_type=jnp.float32)
    g2_ref[...] = h2 * dinv


_t2_call = pl.pallas_call(
    _t2_body,
    out_shape=jax.ShapeDtypeStruct((N_NODES, D_HID), jnp.float32),
)


def _t3_body(accp_ref, g2_ref, dinv_ref, b2_ref, out_ref):
    acc = (accp_ref[0, pl.ds(0, N_NODES), :] +
           accp_ref[1, pl.ds(0, N_NODES), :])
    y = dinv_ref[...] * (acc + g2_ref[...]) + b2_ref[...]
    col = lax.broadcasted_iota(jnp.int32, (N_NODES, D_HID), 1)
    mask = col < N_CLASSES
    z = jnp.where(mask, y, -1e30)
    m = jnp.max(z, axis=1, keepdims=True)
    e = jnp.where(mask, jnp.exp(z - m), 0.0)
    ssum = jnp.sum(e, axis=1, keepdims=True)
    out_ref[...] = z - (m + jnp.log(ssum))


_t3_call = pl.pallas_call(
    _t3_body,
    out_shape=jax.ShapeDtypeStruct((N_NODES, D_HID), jnp.float32),
)


def kernel(x, edge_index, W1, b1, W2, b2):
    ei = edge_index.astype(jnp.int32)
    src = ei[0].reshape(NW, EPT)
    dst = ei[1].reshape(NW, EPT)
    npad = E_TILE - EPT
    pad_src = jnp.zeros((NW, npad), jnp.int32)
    pad_dst = jnp.broadcast_to(
        N_NODES + (jnp.arange(npad, dtype=jnp.int32) % CHUNK), (NW, npad))
    srcp = jnp.concatenate([src, pad_src], 1).reshape(NW, NCHUNK, CHUNK)
    dstp = jnp.concatenate([dst, pad_dst], 1).reshape(NW, NCHUNK, CHUNK)

    degp = _deg_kernel(dstp)
    degp = degp.reshape(NC, N_PAD)[:, :N_NODES].reshape(NC, N_NODES, 1)
    g1, dinv = _t1_call(x, W1, degp)
    accp1 = _agg_kernel(srcp, dstp, g1)
    w2p = jnp.pad(W2, ((0, 0), (0, D_HID - N_CLASSES)))
    g2 = _t2_call(accp1, g1, dinv, b1.reshape(1, D_HID), w2p)
    accp2 = _agg_kernel(srcp, dstp, g2)
    b2p = jnp.pad(b2, (0, D_HID - N_CLASSES)).reshape(1, D_HID)
    out16 = _t3_call(accp2, g2, dinv, b2p)
    return out16[:, :N_CLASSES]


# speedup vs baseline: 59.6994x; 1.4192x over previous
import functools

import jax
import jax.numpy as jnp
from jax import lax
from jax.experimental import pallas as pl
from jax.experimental.pallas import tpu as pltpu
from jax.experimental.pallas import tpu_sc as plsc

N_NODES = 10000
E_TOTAL = 320000
D_FEAT = 128
D_HID = 16
N_CLASSES = 10

NC, NS, LANES = 2, 16, 16
NW = NC * NS
EPT = E_TOTAL // NW
CHUNK = 128
NCHUNK = 80
E_TILE = NCHUNK * CHUNK
N_PAD = 10240
DZ = N_PAD // NS
K = 5
NGROUP = NCHUNK // K
NSETS = 3

_mesh = plsc.VectorSubcoreMesh(core_axis_name="c", subcore_axis_name="s")


@functools.partial(
    pl.kernel,
    mesh=_mesh,
    out_type=jax.ShapeDtypeStruct((NC, 1, N_PAD), jnp.float32),
    scratch_types=[
        pltpu.VMEM((NCHUNK, CHUNK), jnp.int32),
        pltpu.VMEM((CHUNK,), jnp.float32),
        pltpu.VMEM((DZ,), jnp.float32),
        pltpu.VMEM_SHARED((N_PAD,), jnp.float32),
    ],
)
def _deg_kernel(dst_hbm, out_hbm, idx_v, ones_v, zb_v, acc_sh):
    c = lax.axis_index("c")
    s = lax.axis_index("s")
    wid = s * NC + c
    pltpu.sync_copy(dst_hbm.at[wid], idx_v)
    one16 = jnp.ones((LANES,), jnp.float32)
    zero16 = jnp.zeros((LANES,), jnp.float32)
    for i in range(CHUNK // LANES):
        ones_v[pl.ds(i * LANES, LANES)] = one16
    for i in range(DZ // LANES):
        zb_v[pl.ds(i * LANES, LANES)] = zero16
    pltpu.sync_copy(zb_v, acc_sh.at[pl.ds(s * DZ, DZ)])
    plsc.subcore_barrier()

    def body(j, carry):
        pltpu.sync_copy(ones_v, acc_sh.at[idx_v.at[j]], add=True)
        return carry

    lax.fori_loop(0, NCHUNK, body, 0)
    plsc.subcore_barrier()
    pltpu.sync_copy(acc_sh.at[pl.ds(s * DZ, DZ)],
                    out_hbm.at[c, 0, pl.ds(s * DZ, DZ)])


@functools.partial(
    pl.kernel,
    mesh=_mesh,
    compiler_params=pltpu.CompilerParams(use_tc_tiling_on_sc=False),
    out_type=jax.ShapeDtypeStruct((NC, N_PAD, D_HID), jnp.float32),
    scratch_types=[
        pltpu.VMEM((NCHUNK, CHUNK), jnp.int32),
        pltpu.VMEM((NCHUNK, CHUNK), jnp.int32),
        pltpu.VMEM((K * CHUNK, D_HID), jnp.float32),
        pltpu.VMEM((K * CHUNK, D_HID), jnp.float32),
        pltpu.VMEM((K * CHUNK, D_HID), jnp.float32),
        pltpu.VMEM((DZ, D_HID), jnp.float32),
        pltpu.VMEM_SHARED((N_PAD, D_HID), jnp.float32),
        pltpu.VMEM_SHARED((N_PAD, D_HID), jnp.float32),
        pltpu.SemaphoreType.DMA,
        pltpu.SemaphoreType.DMA,
        pltpu.SemaphoreType.DMA,
        pltpu.SemaphoreType.DMA,
        pltpu.SemaphoreType.DMA,
        pltpu.SemaphoreType.DMA,
    ],
)
def _agg_kernel(src_hbm, dst_hbm, g_hbm, out_hbm,
                sidx, didx, buf0, buf1, buf2, zb, acc_sh, g_sh,
                gsem0, gsem1, gsem2, ssem0, ssem1, ssem2):
    c = lax.axis_index("c")
    s = lax.axis_index("s")
    wid = s * NC + c
    bufs = (buf0, buf1, buf2)
    gsems = (gsem0, gsem1, gsem2)
    ssems = (ssem0, ssem1, ssem2)

    pltpu.sync_copy(src_hbm.at[wid], sidx)
    pltpu.sync_copy(dst_hbm.at[wid], didx)
    zero16 = jnp.zeros((LANES,), jnp.float32)

    def zbody(i, carry):
        zb[i, :] = zero16
        return carry

    lax.fori_loop(0, DZ, zbody, 0)
    pltpu.sync_copy(zb, acc_sh.at[pl.ds(s * DZ, DZ)])
    pltpu.sync_copy(g_hbm.at[pl.ds(s * DZ, DZ)], g_sh.at[pl.ds(s * DZ, DZ)])
    plsc.subcore_barrier()

    def fire_gathers(g, st):
        for k in range(K):
            pltpu.async_copy(g_sh.at[sidx.at[g * K + k]],
                             bufs[st].at[pl.ds(k * CHUNK, CHUNK)], gsems[st])

    def wait_gathers(st):
        for _ in range(K):
            pltpu.make_async_copy(g_sh.at[sidx.at[0]],
                                  bufs[st].at[pl.ds(0, CHUNK)],
                                  gsems[st]).wait()

    def fire_scatters(g, st):
        for k in range(K):
            pltpu.async_copy(bufs[st].at[pl.ds(k * CHUNK, CHUNK)],
                             acc_sh.at[didx.at[g * K + k]], ssems[st],
                             add=True)

    def wait_scatters(st):
        for _ in range(K):
            pltpu.make_async_copy(bufs[st].at[pl.ds(0, CHUNK)],
                                  acc_sh.at[didx.at[0]], ssems[st]).wait()

    def part(g, st, drain_next, fire_next):
        if drain_next:
            wait_scatters((st + 1) % NSETS)
        if fire_next:
            fire_gathers(g + 1, (st + 1) % NSETS)
        wait_gathers(st)
        fire_scatters(g, st)

    fire_gathers(0, 0)
    part(0, 0, drain_next=False, fire_next=True)
    part(1, 1, drain_next=False, fire_next=True)
    part(2, 2, drain_next=True, fire_next=True)

    def gbody(t, carry):
        g = t * NSETS
        part(g + 0, 0, drain_next=True, fire_next=True)
        part(g + 1, 1, drain_next=True, fire_next=True)
        part(g + 2, 2, drain_next=True, fire_next=True)
        return carry

    lax.fori_loop(1, NGROUP // NSETS, gbody, 0)

    part(NGROUP - 1, 0, drain_next=True, fire_next=False)
    wait_scatters(2)
    wait_scatters(0)

    plsc.subcore_barrier()
    pltpu.sync_copy(acc_sh.at[pl.ds(s * DZ, DZ)],
                    out_hbm.at[c, pl.ds(s * DZ, DZ)])


def _t1_body(x_ref, w1_ref, degp_ref, g1_ref, dinv_ref):
    deg = degp_ref[0] + degp_ref[1] + 1.0
    dinv = lax.rsqrt(deg)
    h = jnp.dot(x_ref[...], w1_ref[...], preferred_element_type=jnp.float32)
    g1_ref[pl.ds(0, N_NODES), :] = h * dinv
    dinv_ref[...] = dinv


_t1_call = pl.pallas_call(
    _t1_body,
    out_shape=[
        jax.ShapeDtypeStruct((N_PAD, D_HID), jnp.float32),
        jax.ShapeDtypeStruct((N_NODES, 1), jnp.float32),
    ],
)


def _t2_body(accp_ref, g1_ref, dinv_ref, b1_ref, w2_ref, g2_ref):
    acc = (accp_ref[0, pl.ds(0, N_NODES), :] +
           accp_ref[1, pl.ds(0, N_NODES), :])
    dinv = dinv_ref[...]
    g1 = g1_ref[pl.ds(0, N_NODES), :]
    z = jnp.maximum(dinv * (acc + g1) + b1_ref[...], 0.0)
    h2 = jnp.dot(z, w2_ref[...], preferred_element_type=jnp.float32)
    g2_ref[pl.ds(0, N_NODES), :] = h2 * dinv


_t2_call = pl.pallas_call(
    _t2_body,
    out_shape=jax.ShapeDtypeStruct((N_PAD, D_HID), jnp.float32),
)


def _t3_body(accp_ref, g2_ref, dinv_ref, b2_ref, out_ref):
    acc = (accp_ref[0, pl.ds(0, N_NODES), :] +
           accp_ref[1, pl.ds(0, N_NODES), :])
    y = dinv_ref[...] * (acc + g2_ref[pl.ds(0, N_NODES), :]) + b2_ref[...]
    col = lax.broadcasted_iota(jnp.int32, (N_NODES, D_HID), 1)
    mask = col < N_CLASSES
    z = jnp.where(mask, y, -1e30)
    m = jnp.max(z, axis=1, keepdims=True)
    e = jnp.where(mask, jnp.exp(z - m), 0.0)
    ssum = jnp.sum(e, axis=1, keepdims=True)
    out_ref[...] = z - (m + jnp.log(ssum))


_t3_call = pl.pallas_call(
    _t3_body,
    out_shape=jax.ShapeDtypeStruct((N_NODES, D_HID), jnp.float32),
)


def kernel(x, edge_index, W1, b1, W2, b2):
    ei = edge_index.astype(jnp.int32)
    src = ei[0].reshape(NW, EPT)
    dst = ei[1].reshape(NW, EPT)
    npad = E_TILE - EPT
    pad_src = jnp.zeros((NW, npad), jnp.int32)
    pad_dst = jnp.broadcast_to(
        N_NODES + (jnp.arange(npad, dtype=jnp.int32) % CHUNK), (NW, npad))
    srcp = jnp.concatenate([src, pad_src], 1).reshape(NW, NCHUNK, CHUNK)
    dstp = jnp.concatenate([dst, pad_dst], 1).reshape(NW, NCHUNK, CHUNK)

    degp = _deg_kernel(dstp)
    degp = degp.reshape(NC, N_PAD)[:, :N_NODES].reshape(NC, N_NODES, 1)
    g1, dinv = _t1_call(x, W1, degp)
    accp1 = _agg_kernel(srcp, dstp, g1)
    w2p = jnp.pad(W2, ((0, 0), (0, D_HID - N_CLASSES)))
    g2 = _t2_call(accp1, g1, dinv, b1.reshape(1, D_HID), w2p)
    accp2 = _agg_kernel(srcp, dstp, g2)
    b2p = jnp.pad(b2, (0, D_HID - N_CLASSES)).reshape(1, D_HID)
    out16 = _t3_call(accp2, g2, dinv, b2p)
    return out16[:, :N_CLASSES]


# speedup vs baseline: 63.3627x; 1.0614x over previous
import functools

import jax
import jax.numpy as jnp
from jax import lax
from jax.experimental import pallas as pl
from jax.experimental.pallas import tpu as pltpu
from jax.experimental.pallas import tpu_sc as plsc

N_NODES = 10000
E_TOTAL = 320000
D_FEAT = 128
D_HID = 16
N_CLASSES = 10

NC, NS, LANES = 2, 16, 16
NW = NC * NS
EPT = E_TOTAL // NW
CHUNK = 128
NCHUNK = 80
E_TILE = NCHUNK * CHUNK
N_PAD = 10240
DZ = N_PAD // NS
K = 5
NGROUP = NCHUNK // K
NSETS = 3

_mesh = plsc.VectorSubcoreMesh(core_axis_name="c", subcore_axis_name="s")


@functools.partial(
    pl.kernel,
    mesh=_mesh,
    compiler_params=pltpu.CompilerParams(use_tc_tiling_on_sc=False),
    out_type=jax.ShapeDtypeStruct((NC, N_PAD), jnp.float32),
    scratch_types=[
        pltpu.VMEM((NCHUNK, 2, CHUNK), jnp.int32),
        pltpu.VMEM((CHUNK,), jnp.float32),
        pltpu.VMEM((DZ,), jnp.float32),
        pltpu.VMEM_SHARED((N_PAD,), jnp.float32),
    ],
)
def _deg_kernel(ei_hbm, out_hbm, idx_v, ones_v, zb_v, acc_sh):
    c = lax.axis_index("c")
    s = lax.axis_index("s")
    wid = s * NC + c
    pltpu.sync_copy(ei_hbm.at[pl.ds(wid * NCHUNK, NCHUNK)], idx_v)
    one16 = jnp.ones((LANES,), jnp.float32)
    zero16 = jnp.zeros((LANES,), jnp.float32)
    for i in range(CHUNK // LANES):
        ones_v[pl.ds(i * LANES, LANES)] = one16
    for i in range(DZ // LANES):
        zb_v[pl.ds(i * LANES, LANES)] = zero16
    pltpu.sync_copy(zb_v, acc_sh.at[pl.ds(s * DZ, DZ)])
    plsc.subcore_barrier()

    def body(j, carry):
        pltpu.sync_copy(ones_v, acc_sh.at[idx_v.at[j, 1]], add=True)
        return carry

    lax.fori_loop(0, NCHUNK, body, 0)
    plsc.subcore_barrier()
    pltpu.sync_copy(acc_sh.at[pl.ds(s * DZ, DZ)],
                    out_hbm.at[c, pl.ds(s * DZ, DZ)])


@functools.partial(
    pl.kernel,
    mesh=_mesh,
    compiler_params=pltpu.CompilerParams(use_tc_tiling_on_sc=False),
    out_type=jax.ShapeDtypeStruct((NC, N_PAD, D_HID), jnp.float32),
    scratch_types=[
        pltpu.VMEM((NCHUNK, 2, CHUNK), jnp.int32),
        pltpu.VMEM((K * CHUNK, D_HID), jnp.float32),
        pltpu.VMEM((K * CHUNK, D_HID), jnp.float32),
        pltpu.VMEM((K * CHUNK, D_HID), jnp.float32),
        pltpu.VMEM((DZ, D_HID), jnp.float32),
        pltpu.VMEM_SHARED((N_PAD, D_HID), jnp.float32),
        pltpu.VMEM_SHARED((N_PAD, D_HID), jnp.float32),
        pltpu.SemaphoreType.DMA,
        pltpu.SemaphoreType.DMA,
        pltpu.SemaphoreType.DMA,
        pltpu.SemaphoreType.DMA,
        pltpu.SemaphoreType.DMA,
        pltpu.SemaphoreType.DMA,
    ],
)
def _agg_kernel(ei_hbm, g_hbm, out_hbm,
                eidx, buf0, buf1, buf2, zb, acc_sh, g_sh,
                gsem0, gsem1, gsem2, ssem0, ssem1, ssem2):
    c = lax.axis_index("c")
    s = lax.axis_index("s")
    wid = s * NC + c
    bufs = (buf0, buf1, buf2)
    gsems = (gsem0, gsem1, gsem2)
    ssems = (ssem0, ssem1, ssem2)

    pltpu.sync_copy(ei_hbm.at[pl.ds(wid * NCHUNK, NCHUNK)], eidx)
    zero16 = jnp.zeros((LANES,), jnp.float32)

    def zbody(i, carry):
        zb[i, :] = zero16
        return carry

    lax.fori_loop(0, DZ, zbody, 0)
    pltpu.sync_copy(zb, acc_sh.at[pl.ds(s * DZ, DZ)])
    pltpu.sync_copy(g_hbm.at[pl.ds(s * DZ, DZ)], g_sh.at[pl.ds(s * DZ, DZ)])
    plsc.subcore_barrier()

    def fire_gathers(g, st):
        for k in range(K):
            pltpu.async_copy(g_sh.at[eidx.at[g * K + k, 0]],
                             bufs[st].at[pl.ds(k * CHUNK, CHUNK)], gsems[st])

    def wait_gathers(st):
        for _ in range(K):
            pltpu.make_async_copy(g_sh.at[eidx.at[0, 0]],
                                  bufs[st].at[pl.ds(0, CHUNK)],
                                  gsems[st]).wait()

    def fire_scatters(g, st):
        for k in range(K):
            pltpu.async_copy(bufs[st].at[pl.ds(k * CHUNK, CHUNK)],
                             acc_sh.at[eidx.at[g * K + k, 1]], ssems[st],
                             add=True)

    def wait_scatters(st):
        for _ in range(K):
            pltpu.make_async_copy(bufs[st].at[pl.ds(0, CHUNK)],
                                  acc_sh.at[eidx.at[0, 1]], ssems[st]).wait()

    def part(g, st, drain_next, fire_next):
        if drain_next:
            wait_scatters((st + 1) % NSETS)
        if fire_next:
            fire_gathers(g + 1, (st + 1) % NSETS)
        wait_gathers(st)
        fire_scatters(g, st)

    fire_gathers(0, 0)
    part(0, 0, drain_next=False, fire_next=True)
    part(1, 1, drain_next=False, fire_next=True)
    part(2, 2, drain_next=True, fire_next=True)

    def gbody(t, carry):
        g = t * NSETS
        part(g + 0, 0, drain_next=True, fire_next=True)
        part(g + 1, 1, drain_next=True, fire_next=True)
        part(g + 2, 2, drain_next=True, fire_next=True)
        return carry

    lax.fori_loop(1, NGROUP // NSETS, gbody, 0)

    part(NGROUP - 1, 0, drain_next=True, fire_next=False)
    wait_scatters(2)
    wait_scatters(0)

    plsc.subcore_barrier()
    pltpu.sync_copy(acc_sh.at[pl.ds(s * DZ, DZ)],
                    out_hbm.at[c, pl.ds(s * DZ, DZ)])


def _t1_body(x_ref, w1_ref, deg_ref, g1_ref, dinv_ref):
    deg = deg_ref[...] + 1.0
    dinv = lax.rsqrt(deg)
    h = jnp.dot(x_ref[...], w1_ref[...], preferred_element_type=jnp.float32)
    g1_ref[pl.ds(0, N_NODES), :] = h * dinv
    dinv_ref[...] = dinv


_t1_call = pl.pallas_call(
    _t1_body,
    out_shape=[
        jax.ShapeDtypeStruct((N_PAD, D_HID), jnp.float32),
        jax.ShapeDtypeStruct((N_NODES, 1), jnp.float32),
    ],
)


def _t2_body(accp_ref, g1_ref, dinv_ref, b1_ref, w2_ref, g2_ref):
    acc = (accp_ref[0, pl.ds(0, N_NODES), :] +
           accp_ref[1, pl.ds(0, N_NODES), :])
    dinv = dinv_ref[...]
    g1 = g1_ref[pl.ds(0, N_NODES), :]
    z = jnp.maximum(dinv * (acc + g1) + b1_ref[...], 0.0)
    h2 = jnp.dot(z, w2_ref[...], preferred_element_type=jnp.float32)
    g2_ref[pl.ds(0, N_NODES), :] = h2 * dinv


_t2_call = pl.pallas_call(
    _t2_body,
    out_shape=jax.ShapeDtypeStruct((N_PAD, D_HID), jnp.float32),
)


def _t3_body(accp_ref, g2_ref, dinv_ref, b2_ref, out_ref):
    acc = (accp_ref[0, pl.ds(0, N_NODES), :] +
           accp_ref[1, pl.ds(0, N_NODES), :])
    y = dinv_ref[...] * (acc + g2_ref[pl.ds(0, N_NODES), :]) + b2_ref[...]
    col = lax.broadcasted_iota(jnp.int32, (N_NODES, D_HID), 1)
    mask = col < N_CLASSES
    z = jnp.where(mask, y, -1e30)
    m = jnp.max(z, axis=1, keepdims=True)
    e = jnp.where(mask, jnp.exp(z - m), 0.0)
    ssum = jnp.sum(e, axis=1, keepdims=True)
    out_ref[...] = z - (m + jnp.log(ssum))


_t3_call = pl.pallas_call(
    _t3_body,
    out_shape=jax.ShapeDtypeStruct((N_NODES, D_HID), jnp.float32),
)


def kernel(x, edge_index, W1, b1, W2, b2):
    ei = edge_index.astype(jnp.int32)
    npad = NW * E_TILE - E_TOTAL
    pad_src = jnp.zeros((1, npad), jnp.int32)
    pad_dst = (N_NODES
               + (jnp.arange(npad, dtype=jnp.int32) % CHUNK)).reshape(1, npad)
    eip = jnp.concatenate([ei, jnp.concatenate([pad_src, pad_dst], 0)], 1)
    ei3 = eip.reshape(2, NW * NCHUNK, CHUNK).transpose(1, 0, 2)

    degp = _deg_kernel(ei3)
    deg = (degp[0, :N_NODES] + degp[1, :N_NODES]).reshape(N_NODES, 1)
    g1, dinv = _t1_call(x, W1, deg)
    accp1 = _agg_kernel(ei3, g1)
    w2p = jnp.pad(W2, ((0, 0), (0, D_HID - N_CLASSES)))
    g2 = _t2_call(accp1, g1, dinv, b1.reshape(1, D_HID), w2p)
    accp2 = _agg_kernel(ei3, g2)
    b2p = jnp.pad(b2, (0, D_HID - N_CLASSES)).reshape(1, D_HID)
    out16 = _t3_call(accp2, g2, dinv, b2p)
    return out16[:, :N_CLASSES]
